# Initial kernel scaffold; baseline (speedup 1.0000x reference)
import jax
import jax.numpy as jnp
from jax.experimental import pallas as pl


def kernel(inputs):
    raise NotImplementedError("write your pallas kernel here")


# speedup vs baseline: 15.7569x; 15.7569x over previous
import functools

import jax
import jax.numpy as jnp
from jax.experimental import pallas as pl
from jax.experimental.pallas import tpu as pltpu

RATIO = 0.1
INT_MIN = -2147483648


def _sortable(xi):
    return xi ^ ((xi >> 31) & jnp.int32(0x7FFFFFFF))


def _kwta_block(in_ref, out_ref, scratch_ref, *, k):
    x = in_ref[...]
    xi = pltpu.bitcast(x, jnp.int32)
    scratch_ref[...] = _sortable(xi)

    def body(b, t):
        bit = jnp.int32(1) << (jnp.int32(31) - b)
        trial = t | bit
        cand = trial ^ jnp.int32(INT_MIN)
        cnt = jnp.sum((scratch_ref[...] >= cand).astype(jnp.int32), axis=1,
                      keepdims=True)
        return jnp.where(cnt >= k, trial, t)

    t0 = jnp.zeros((x.shape[0], 1), jnp.int32)
    t = jax.lax.fori_loop(0, 32, body, t0)
    thr_bits = _sortable(t ^ jnp.int32(INT_MIN))
    thr = pltpu.bitcast(thr_bits, jnp.float32)
    out_ref[...] = jnp.where(x >= thr, x, jnp.float32(0.0))


def kernel(inputs):
    rows, features = inputs.shape
    k = max(int(round(RATIO * features)), 1)
    block_rows = 16
    grid = rows // block_rows
    return pl.pallas_call(
        functools.partial(_kwta_block, k=k),
        grid=(grid,),
        in_specs=[pl.BlockSpec((block_rows, features), lambda i: (i, 0))],
        out_specs=pl.BlockSpec((block_rows, features), lambda i: (i, 0)),
        out_shape=jax.ShapeDtypeStruct(inputs.shape, inputs.dtype),
        scratch_shapes=[pltpu.VMEM((block_rows, features), jnp.int32)],
    )(inputs)
